# baseline (speedup 1.0000x reference)
import functools

import jax
import jax.numpy as jnp
from jax import lax
from jax.experimental import pallas as pl
from jax.experimental.pallas import tpu as pltpu, tpu_sc as plsc

D = 16
V = 100000
B = 16384
NC = 2
NS = 16
DH = D // NC
B_PER_T = B // NS
LB = B_PER_T // 128
L = 16

NBLK = 48
NCOL = NBLK * 128
HB = NBLK // 2
MAIN = NS * NCOL
TAILN = V - MAIN

_mesh = plsc.VectorSubcoreMesh(core_axis_name="c", subcore_axis_name="s")


@functools.partial(
    pl.kernel,
    mesh=_mesh,
    out_type=jax.ShapeDtypeStruct((B * D // 128, 128), jnp.float32),
    scratch_types=[
        pltpu.VMEM((B_PER_T,), jnp.int32),
        pltpu.VMEM((DH * B_PER_T,), jnp.int32),
        pltpu.VMEM((LB, DH, 128), jnp.float32),
        pltpu.VMEM((DH, HB * 128), jnp.float32),
        pltpu.VMEM((2 * DH * 1024,), jnp.float32),
        pltpu.VMEM_SHARED((DH * V + D * TAILN,), jnp.float32),
        pltpu.SemaphoreType.DMA,
        pltpu.SemaphoreType.DMA,
        pltpu.SemaphoreType.DMA,
        pltpu.SemaphoreType.DMA,
    ],
    compiler_params=pltpu.CompilerParams(
        disable_bounds_checks=True,
        disable_semaphore_checks=True,
        skip_device_barrier=True,
    ),
)
def _gather_kernel(idx_hbm, table_t_hbm, tail_hbm, out_hbm, idx_v, gidx_v,
                   buf_v, slab_v, lin_v, tbl_sp, isem, ssem, osem, gsem):
    cid = lax.axis_index("c")
    sid = lax.axis_index("s")

    def stage_in(ph):
        return [
            pltpu.async_copy(
                table_t_hbm.at[
                    pl.ds(cid * DH, DH),
                    pl.ds((sid * NBLK + ph * HB + k) * 128, 128),
                ],
                slab_v.at[:, pl.ds(k * 128, 128)],
                isem,
            )
            for k in range(HB)
        ]

    fetch = stage_in(0)

    @pl.when(sid == 0)
    def _tail():
        pltpu.sync_copy(tail_hbm, tbl_sp.at[pl.ds(DH * V, D * TAILN)])

    pltpu.sync_copy(idx_hbm.at[pl.ds(sid * B_PER_T, B_PER_T)], idx_v)

    def gbody(c, _):
        iv = idx_v[pl.ds(c * L, L)]
        tail_base = DH * V + cid * DH * TAILN - MAIN
        for dd in range(DH):
            gidx_v[pl.ds(dd * B_PER_T + c * L, L)] = jnp.where(
                iv >= MAIN,
                iv + (tail_base + dd * TAILN),
                iv + dd * V,
            )
        return 0

    lax.fori_loop(0, B_PER_T // L, gbody, 0)

    def make_shuffle(ch2, ch):
        base = (ch % 2) * DH * 1024

        def sbody(b, _):
            for r in range(DH):
                for j in range(128 // L):
                    lin_v[pl.ds(base + r * 1024 + b * 128 + j * L, L)] = (
                        slab_v[r, pl.ds(ch2 * 1024 + b * 128 + j * L, L)]
                    )
            return 0

        lax.fori_loop(0, 8, sbody, 0)

    pending = [None, None]
    for ph in range(2):
        for c in fetch:
            c.wait()
        for ch2 in range(3):
            ch = ph * 3 + ch2
            if pending[ch % 2] is not None:
                for c in pending[ch % 2]:
                    c.wait()
            make_shuffle(ch2, ch)
            if ph == 0 and ch2 == 2:
                fetch = stage_in(1)
            pending[ch % 2] = [
                pltpu.async_copy(
                    lin_v.at[pl.ds((ch % 2) * DH * 1024 + r * 1024, 1024)],
                    tbl_sp.at[pl.ds(r * V + sid * NCOL + ch * 1024, 1024)],
                    ssem,
                )
                for r in range(DH)
            ]
    for p in pending:
        if p is not None:
            for c in p:
                c.wait()
    plsc.subcore_barrier()

    copies = [
        pltpu.async_copy(
            tbl_sp.at[gidx_v.at[pl.ds(dd * B_PER_T + lb * 128, 128)]],
            buf_v.at[lb, dd],
            gsem,
        )
        for dd in range(DH)
        for lb in range(LB)
    ]
    for c in copies:
        c.wait()
    outs = []
    for lb in range(LB):
        bb = cid * 128 + sid * LB + lb
        outs.append(
            pltpu.async_copy(
                buf_v.at[lb],
                out_hbm.at[pl.ds(bb * DH, DH), :],
                osem,
            )
        )
    for o in outs:
        o.wait()


def kernel(task_id, table):
    tail = table.T[:, MAIN:].reshape(-1)
    out2048 = _gather_kernel(task_id.astype(jnp.int32), table.T, tail)
    return (
        out2048.reshape(NC, B // 128, DH, 128)
        .transpose(1, 3, 0, 2)
        .reshape(B, D)
    )


# speedup vs baseline: 1.2843x; 1.2843x over previous
import functools

import jax
import jax.numpy as jnp
from jax import lax
from jax.experimental import pallas as pl
from jax.experimental.pallas import tpu as pltpu, tpu_sc as plsc

D = 16
V = 100000
B = 16384
NC = 2
NS = 16
DH = D // NC
B_PER_T = B // NS
LB = B_PER_T // 128
L = 16

_mesh = plsc.VectorSubcoreMesh(core_axis_name="c", subcore_axis_name="s")


@functools.partial(
    pl.kernel,
    mesh=_mesh,
    out_type=jax.ShapeDtypeStruct((B * D // 128, 128), jnp.float32),
    scratch_types=[
        pltpu.VMEM((B_PER_T,), jnp.int32),
        pltpu.VMEM((DH * B_PER_T,), jnp.int32),
        pltpu.VMEM((LB, DH, 128), jnp.float32),
        pltpu.VMEM_SHARED((DH * V,), jnp.float32),
        pltpu.SemaphoreType.DMA,
        pltpu.SemaphoreType.DMA,
    ],
    compiler_params=pltpu.CompilerParams(
        use_tc_tiling_on_sc=False,
        disable_bounds_checks=True,
        disable_semaphore_checks=True,
        skip_device_barrier=True,
    ),
)
def _gather_kernel(idx_hbm, table_t_hbm, out_hbm, idx_v, gidx_v, buf_v,
                   tbl_sp, ssem, gsem):
    cid = lax.axis_index("c")
    sid = lax.axis_index("s")
    row = cid * DH + sid // 2
    half = (sid % 2) * (V // 2)
    stage = pltpu.async_copy(
        table_t_hbm.at[row, pl.ds(half, V // 2)],
        tbl_sp.at[pl.ds((sid // 2) * V + half, V // 2)],
        ssem,
    )
    pltpu.sync_copy(idx_hbm.at[pl.ds(sid * B_PER_T, B_PER_T)], idx_v)

    def body(c, _):
        iv = idx_v[pl.ds(c * L, L)]
        for dd in range(DH):
            gidx_v[pl.ds(dd * B_PER_T + c * L, L)] = iv + dd * V
        return 0

    lax.fori_loop(0, B_PER_T // L, body, 0)
    stage.wait()
    plsc.subcore_barrier()
    copies = [
        pltpu.async_copy(
            tbl_sp.at[gidx_v.at[pl.ds(dd * B_PER_T + lb * 128, 128)]],
            buf_v.at[lb, dd],
            gsem,
        )
        for dd in range(DH)
        for lb in range(LB)
    ]
    for c in copies:
        c.wait()
    outs = []
    for lb in range(LB):
        bb = cid * 128 + sid * LB + lb
        outs.append(
            pltpu.async_copy(
                buf_v.at[lb],
                out_hbm.at[pl.ds(bb * DH, DH), :],
                ssem,
            )
        )
    for o in outs:
        o.wait()


def kernel(task_id, table):
    out2048 = _gather_kernel(task_id.astype(jnp.int32), table.T)
    return (
        out2048.reshape(NC, B // 128, DH, 128)
        .transpose(1, 3, 0, 2)
        .reshape(B, D)
    )
